# baseline (speedup 1.0000x reference)
import functools

import jax
import jax.numpy as jnp
from jax import lax
from jax.experimental import pallas as pl
from jax.experimental.pallas import tpu as pltpu
from jax.experimental.pallas import tpu_sc as plsc

_D = 64
_B = 4096
_H = 200
_V = 1000000
_L = 128
_NW = 32

_mesh = plsc.VectorSubcoreMesh(core_axis_name="c", subcore_axis_name="s")


@functools.partial(
    pl.kernel,
    out_type=jax.ShapeDtypeStruct((_H, _D, _B), jnp.float32),
    mesh=_mesh,
    scratch_types=[
        pltpu.VMEM((_H, _L), jnp.int32),
        pltpu.VMEM((_L,), jnp.int32),
        pltpu.VMEM((_L,), jnp.int32),
        pltpu.VMEM((_L,), jnp.int32),
        pltpu.VMEM((_L,), jnp.int32),
        pltpu.VMEM((_L, _L), jnp.float32),
        pltpu.VMEM((_L, _L), jnp.float32),
        pltpu.VMEM((_D, _L), jnp.float32),
        pltpu.VMEM((_D, _L), jnp.float32),
        pltpu.SemaphoreType.DMA,
        pltpu.SemaphoreType.DMA,
        pltpu.SemaphoreType.DMA,
        pltpu.SemaphoreType.DMA,
    ],
    compiler_params=pltpu.CompilerParams(
        use_tc_tiling_on_sc=True, needs_layout_passes=False),
)
def _gather_t(idx_hbm, tab_hbm, out_hbm, idx_v, i20, i21, hb0, hb1,
              rows0, rows1, til0, til1, sg0, sg1, sw0, sw1):
    wid = lax.axis_index("s") * 2 + lax.axis_index("c")
    b0 = wid * _L

    pltpu.sync_copy(idx_hbm.at[:, pl.ds(b0, _L)], idx_v)

    i2s = (i20, i21)
    hbs = (hb0, hb1)
    rows = (rows0, rows1)
    tils = (til0, til1)
    sgs = (sg0, sg1)
    sws = (sw0, sw1)

    lane = lax.iota(jnp.int32, 16)
    bvecs = [lane + (c * 16) for c in range(8)]

    def prep(h, p):
        for c in range(8):
            iv = idx_v[h, pl.ds(c * 16, 16)]
            i2s[p][pl.ds(c * 16, 16)] = lax.shift_right_logical(iv, 1)
            hbs[p][pl.ds(c * 16, 16)] = lax.shift_left(iv & 1, 6)

    def g_desc(p):
        return pltpu.make_async_copy(tab_hbm.at[i2s[p]], rows[p], sgs[p])

    def w_desc(h, p):
        return pltpu.make_async_copy(
            tils[p], out_hbm.at[h, :, pl.ds(b0, _L)], sws[p])

    def transpose(p):
        hvecs = [hbs[p][pl.ds(c * 16, 16)] for c in range(8)]
        for d in range(_D):
            for c in range(8):
                vals = plsc.load_gather(rows[p], [bvecs[c], hvecs[c] + d])
                tils[p][d, pl.ds(c * 16, 16)] = vals

    prep(0, 0)
    g_desc(0).start()
    prep(1, 1)
    g_desc(1).start()

    def body(j, carry):
        for p in range(2):
            h = j * 2 + p
            g_desc(p).wait()

            @pl.when(h >= 2)
            def _():
                w_desc(h - 2, p).wait()

            transpose(p)
            w_desc(h, p).start()

            @pl.when(h + 2 < _H)
            def _():
                prep(h + 2, p)
                g_desc(p).start()

        return carry

    lax.fori_loop(0, _H // 2, body, 0)
    w_desc(_H - 2, 0).wait()
    w_desc(_H - 1, 1).wait()


def kernel(x, table):
    xt = x.astype(jnp.int32).T
    tab2 = jnp.reshape(table, (_V // 2, _D * 2))
    out_t = _gather_t(xt, tab2)
    return jnp.transpose(out_t, (2, 0, 1))


# speedup vs baseline: 1.9054x; 1.9054x over previous
import functools

import jax
import jax.numpy as jnp
from jax import lax
from jax.experimental import pallas as pl
from jax.experimental.pallas import tpu as pltpu
from jax.experimental.pallas import tpu_sc as plsc

_D = 64
_B = 4096
_H = 200
_V = 1000000
_L = 128
_NW = 32

_mesh = plsc.VectorSubcoreMesh(core_axis_name="c", subcore_axis_name="s")


@functools.partial(
    pl.kernel,
    out_type=jax.ShapeDtypeStruct((_H, _D, _B), jnp.float32),
    mesh=_mesh,
    scratch_types=[
        pltpu.VMEM((_H, _L), jnp.int32),
        pltpu.VMEM((_L,), jnp.int32),
        pltpu.VMEM((_L,), jnp.int32),
        pltpu.VMEM((_L,), jnp.int32),
        pltpu.VMEM((_L,), jnp.int32),
        pltpu.VMEM((_L, _L), jnp.float32),
        pltpu.VMEM((_L, _L), jnp.float32),
        pltpu.VMEM((_D, _L), jnp.float32),
        pltpu.VMEM((_D, _L), jnp.float32),
        pltpu.SemaphoreType.DMA,
        pltpu.SemaphoreType.DMA,
        pltpu.SemaphoreType.DMA,
        pltpu.SemaphoreType.DMA,
    ],
    compiler_params=pltpu.CompilerParams(
        use_tc_tiling_on_sc=True, needs_layout_passes=False),
)
def _gather_t(idx_hbm, tab_hbm, out_hbm, idx_v, i20, i21, hb0, hb1,
              rows0, rows1, til0, til1, sg0, sg1, sw0, sw1):
    wid = lax.axis_index("s") * 2 + lax.axis_index("c")
    b0 = wid * _L

    pltpu.sync_copy(idx_hbm.at[:, pl.ds(b0, _L)], idx_v)

    i2s = (i20, i21)
    hbs = (hb0, hb1)
    rows = (rows0, rows1)
    tils = (til0, til1)
    sgs = (sg0, sg1)
    sws = (sw0, sw1)

    lane = lax.iota(jnp.int32, 16)
    bvecs = [lane + (c * 16) for c in range(8)]

    def prep(h, p):
        for c in range(8):
            iv = idx_v[h, pl.ds(c * 16, 16)]
            i2s[p][pl.ds(c * 16, 16)] = lax.shift_right_logical(iv, 1)
            hbs[p][pl.ds(c * 16, 16)] = lax.shift_left(iv & 1, 6)

    def g_desc(p):
        return pltpu.make_async_copy(tab_hbm.at[i2s[p]], rows[p], sgs[p])

    def w_desc(h, p):
        return pltpu.make_async_copy(
            tils[p], out_hbm.at[h, :, pl.ds(b0, _L)], sws[p])

    def transpose(p):
        hvecs = [hbs[p][pl.ds(c * 16, 16)] for c in range(8)]

        def per_s(s, carry):
            rot = (lane + s) & 15
            for m in range(4):
                d_pure = rot + (16 * m)
                for c in range(8):
                    v = plsc.load_gather(
                        rows[p], [bvecs[c], hvecs[c] + d_pure])
                    plsc.store_scatter(tils[p], [d_pure, bvecs[c]], v)
            return carry

        lax.fori_loop(0, 16, per_s, 0)

    prep(0, 0)
    g_desc(0).start()
    prep(1, 1)
    g_desc(1).start()

    def body(j, carry):
        for p in range(2):
            h = j * 2 + p
            g_desc(p).wait()

            @pl.when(h >= 2)
            def _():
                w_desc(h - 2, p).wait()

            transpose(p)
            w_desc(h, p).start()

            @pl.when(h + 2 < _H)
            def _():
                prep(h + 2, p)
                g_desc(p).start()

        return carry

    lax.fori_loop(0, _H // 2, body, 0)
    w_desc(_H - 2, 0).wait()
    w_desc(_H - 1, 1).wait()


def kernel(x, table):
    xt = x.astype(jnp.int32).T
    tab2 = jnp.reshape(table, (_V // 2, _D * 2))
    out_t = _gather_t(xt, tab2)
    return jnp.transpose(out_t, (2, 0, 1))


# speedup vs baseline: 2.2891x; 1.2014x over previous
import functools

import jax
import jax.numpy as jnp
from jax import lax
from jax.experimental import pallas as pl
from jax.experimental.pallas import tpu as pltpu
from jax.experimental.pallas import tpu_sc as plsc

_D = 64
_B = 4096
_H = 200
_V = 1000000
_L = 128
_NW = 32

_mesh = plsc.VectorSubcoreMesh(core_axis_name="c", subcore_axis_name="s")


@functools.partial(
    pl.kernel,
    out_type=jax.ShapeDtypeStruct((_H, _D, _B), jnp.float32),
    mesh=_mesh,
    scratch_types=[
        pltpu.VMEM((_H, _L), jnp.int32),
        pltpu.VMEM((_L,), jnp.int32),
        pltpu.VMEM((_L,), jnp.int32),
        pltpu.VMEM((_L,), jnp.int32),
        pltpu.VMEM((_L,), jnp.int32),
        pltpu.VMEM((_L, _L), jnp.float32),
        pltpu.VMEM((_L, _L), jnp.float32),
        pltpu.VMEM((_D, _L), jnp.float32),
        pltpu.VMEM((_D, _L), jnp.float32),
        pltpu.SemaphoreType.DMA,
        pltpu.SemaphoreType.DMA,
        pltpu.SemaphoreType.DMA,
        pltpu.SemaphoreType.DMA,
    ],
    compiler_params=pltpu.CompilerParams(
        use_tc_tiling_on_sc=True, needs_layout_passes=False),
)
def _gather_t(idx_hbm, tab_hbm, out_hbm, idx_v, i20, i21, hb0, hb1,
              rows0, rows1, til0, til1, sg0, sg1, sw0, sw1):
    wid = lax.axis_index("s") * 2 + lax.axis_index("c")
    b0 = wid * _L

    pltpu.sync_copy(idx_hbm.at[:, pl.ds(b0, _L)], idx_v)

    i2s = (i20, i21)
    hbs = (hb0, hb1)
    rows = (rows0, rows1)
    tils = (til0, til1)
    sgs = (sg0, sg1)
    sws = (sw0, sw1)

    lane = lax.iota(jnp.int32, 16)
    bvecs = [lane + (c * 16) for c in range(8)]

    def prep(h, p):
        for c in range(8):
            iv = idx_v[h, pl.ds(c * 16, 16)]
            i2s[p][pl.ds(c * 16, 16)] = lax.shift_right_logical(iv, 1)
            hbs[p][pl.ds(c * 16, 16)] = lax.shift_left(iv & 1, 6)

    def g_desc(p):
        return pltpu.make_async_copy(tab_hbm.at[i2s[p]], rows[p], sgs[p])

    def w_desc(h, p):
        return pltpu.make_async_copy(
            tils[p], out_hbm.at[h, :, pl.ds(b0, _L)], sws[p])

    def transpose(p):
        hvecs = [hbs[p][pl.ds(c * 16, 16)] for c in range(8)]

        @plsc.parallel_loop(0, 16, unroll=2)
        def per_s(s):
            rot = (lane + s) & 15
            for m in range(4):
                d_pure = rot + (16 * m)
                for c in range(8):
                    v = plsc.load_gather(
                        rows[p], [bvecs[c], hvecs[c] + d_pure])
                    plsc.store_scatter(tils[p], [d_pure, bvecs[c]], v)

    prep(0, 0)
    g_desc(0).start()
    prep(1, 1)
    g_desc(1).start()

    def body(j, carry):
        for p in range(2):
            h = j * 2 + p
            g_desc(p).wait()

            @pl.when(h >= 2)
            def _():
                w_desc(h - 2, p).wait()

            transpose(p)
            w_desc(h, p).start()

            @pl.when(h + 2 < _H)
            def _():
                prep(h + 2, p)
                g_desc(p).start()

        return carry

    lax.fori_loop(0, _H // 2, body, 0)
    w_desc(_H - 2, 0).wait()
    w_desc(_H - 1, 1).wait()


def kernel(x, table):
    xt = x.astype(jnp.int32).T
    tab2 = jnp.reshape(table, (_V // 2, _D * 2))
    out_t = _gather_t(xt, tab2)
    return jnp.transpose(out_t, (2, 0, 1))


# speedup vs baseline: 3.8360x; 1.6757x over previous
import functools

import jax
import jax.numpy as jnp
from jax import lax
from jax.experimental import pallas as pl
from jax.experimental.pallas import tpu as pltpu
from jax.experimental.pallas import tpu_sc as plsc

_D = 64
_B = 4096
_H = 200
_V = 1000000
_L = 128
_NW = 32

_mesh = plsc.VectorSubcoreMesh(core_axis_name="c", subcore_axis_name="s")


@functools.partial(
    pl.kernel,
    out_type=jax.ShapeDtypeStruct((_H, _D, _B), jnp.float32),
    mesh=_mesh,
    scratch_types=[
        pltpu.VMEM((_H, _L), jnp.int32),
        pltpu.VMEM((_L,), jnp.int32),
        pltpu.VMEM((_L,), jnp.int32),
        pltpu.VMEM((_L,), jnp.int32),
        pltpu.VMEM((_L,), jnp.int32),
        pltpu.VMEM((_L, _L), jnp.float32),
        pltpu.VMEM((_L, _L), jnp.float32),
        pltpu.VMEM((_D, _L), jnp.float32),
        pltpu.VMEM((_D, _L), jnp.float32),
        pltpu.SemaphoreType.DMA,
        pltpu.SemaphoreType.DMA,
        pltpu.SemaphoreType.DMA,
        pltpu.SemaphoreType.DMA,
    ],
    compiler_params=pltpu.CompilerParams(
        use_tc_tiling_on_sc=True, needs_layout_passes=False),
)
def _gather_t(idx_hbm, tab_hbm, out_hbm, idx_v, i20, i21, hb0, hb1,
              rows0, rows1, til0, til1, sg0, sg1, sw0, sw1):
    wid = lax.axis_index("s") * 2 + lax.axis_index("c")
    b0 = wid * _L

    pltpu.sync_copy(idx_hbm.at[:, pl.ds(b0, _L)], idx_v)

    i2s = (i20, i21)
    hbs = (hb0, hb1)
    rows = (rows0, rows1)
    tils = (til0, til1)
    sgs = (sg0, sg1)
    sws = (sw0, sw1)

    lane = lax.iota(jnp.int32, 16)
    bvecs = [lane + (c * 16) for c in range(8)]

    def prep(h, p):
        for c in range(8):
            iv = idx_v[h, pl.ds(c * 16, 16)]
            i2s[p][pl.ds(c * 16, 16)] = lax.shift_right_logical(iv, 1)
            hbs[p][pl.ds(c * 16, 16)] = lax.shift_left(iv & 1, 6)

    def g_desc(p):
        return pltpu.make_async_copy(tab_hbm.at[i2s[p]], rows[p], sgs[p])

    def w_desc(h, p):
        return pltpu.make_async_copy(
            tils[p], out_hbm.at[h, :, pl.ds(b0, _L)], sws[p])

    def transpose(p):
        hvecs = [hbs[p][pl.ds(c * 16, 16)] for c in range(8)]

        @plsc.parallel_loop(0, 16, unroll=2)
        def per_s(s):
            rot = (lane + s) & 15
            for m in range(4):
                d_pure = rot + (16 * m)
                for c in range(8):
                    v = plsc.load_gather(
                        rows[p], [bvecs[c], hvecs[c] + d_pure])
                    plsc.store_scatter(tils[p], [d_pure, bvecs[c]], v)

    prep(0, 0)
    g_desc(0).start()
    prep(1, 1)
    g_desc(1).start()

    def body(j, carry):
        for p in range(2):
            h = j * 2 + p
            g_desc(p).wait()

            @pl.when(h >= 2)
            def _():
                w_desc(h - 2, p).wait()

            transpose(p)
            w_desc(h, p).start()

            @pl.when(h + 2 < _H)
            def _():
                prep(h + 2, p)
                g_desc(p).start()

        return carry

    lax.fori_loop(0, _H // 2, body, 0)
    w_desc(_H - 2, 0).wait()
    w_desc(_H - 1, 1).wait()


_NBLK = 7812
_BPW = 246


@functools.partial(
    pl.kernel,
    out_type=jax.ShapeDtypeStruct((_V // 2, 2 * _D), jnp.float32),
    mesh=_mesh,
    scratch_types=[
        pltpu.VMEM((_D, _L), jnp.float32),
        pltpu.VMEM((_D, _L), jnp.float32),
        pltpu.VMEM((_D, _L), jnp.float32),
        pltpu.VMEM((_D, _L), jnp.float32),
        pltpu.VMEM((32, _L), jnp.float32),
        pltpu.SemaphoreType.DMA,
        pltpu.SemaphoreType.DMA,
        pltpu.SemaphoreType.DMA,
        pltpu.SemaphoreType.DMA,
    ],
    compiler_params=pltpu.CompilerParams(
        use_tc_tiling_on_sc=True, needs_layout_passes=False),
)
def _pairify(tabt_hbm, tail_hbm, pairs_hbm, st0, st1, pb0, pb1, tlv,
             si0, si1, so0, so1):
    wid = lax.axis_index("s") * 2 + lax.axis_index("c")

    sts = (st0, st1)
    pbs = (pb0, pb1)
    sis = (si0, si1)
    sos = (so0, so1)

    lane = lax.iota(jnp.int32, 16)
    vvecs = [lane + (v0 * 16) for v0 in range(8)]
    qvecs = [lax.shift_right_logical(vv, 1) for vv in vvecs]
    ovecs = [lax.shift_left(vv & 1, 6) for vv in vvecs]

    def in_desc(c, p):
        return pltpu.make_async_copy(
            tabt_hbm.at[:, pl.ds(c * _L, _L)], sts[p], sis[p])

    def out_desc(c, p):
        return pltpu.make_async_copy(
            pbs[p], pairs_hbm.at[pl.ds(c * _D, _D)], sos[p])

    def fire_in(i, p):
        c = wid + i * _NW

        @pl.when(c < _NBLK)
        def _():
            in_desc(c, p).start()

    def wait_in(i, p):
        c = wid + i * _NW

        @pl.when(c < _NBLK)
        def _():
            in_desc(c, p).wait()

    def fire_out(i, p):
        c = wid + i * _NW

        @pl.when(c < _NBLK)
        def _():
            out_desc(c, p).start()

    def wait_out(i, p):
        c = wid + i * _NW

        @pl.when(c < _NBLK)
        def _():
            out_desc(c, p).wait()

    def transpose(p):
        @plsc.parallel_loop(0, 16, unroll=2)
        def per_s(s):
            rot = (lane + s) & 15
            for m in range(4):
                d_pure = rot + (16 * m)
                for v0 in range(8):
                    v = plsc.load_gather(sts[p], [d_pure, vvecs[v0]])
                    plsc.store_scatter(
                        pbs[p], [qvecs[v0], ovecs[v0] + d_pure], v)

    @pl.when(wid == 0)
    def _():
        pltpu.sync_copy(tail_hbm, tlv)
        pltpu.sync_copy(tlv, pairs_hbm.at[pl.ds(_V // 2 - 32, 32)])

    fire_in(0, 0)
    fire_in(1, 1)

    def body(j, carry):
        for p in range(2):
            i = j * 2 + p
            c = wid + i * _NW

            @pl.when(c < _NBLK)
            def _():
                wait_in(i, p)

                @pl.when(i >= 2)
                def _():
                    wait_out(i - 2, p)

                transpose(p)
                fire_out(i, p)
                fire_in(i + 2, p)

        return carry

    lax.fori_loop(0, _BPW // 2, body, 0)

    for i in (_BPW - 4, _BPW - 3, _BPW - 2, _BPW - 1):
        c = wid + i * _NW

        @pl.when((c < _NBLK) & (c + 2 * _NW >= _NBLK))
        def _():
            wait_out(i, i % 2)


def kernel(x, table):
    xt = x.astype(jnp.int32).T
    tabt = table.T
    tailp = jnp.reshape(
        lax.slice(table, (_V - _D, 0), (_V, _D)), (32, 2 * _D))
    tab2 = _pairify(tabt, tailp)
    out_t = _gather_t(xt, tab2)
    return jnp.transpose(out_t, (2, 0, 1))


# speedup vs baseline: 3.9353x; 1.0259x over previous
import functools

import jax
import jax.numpy as jnp
from jax import lax
from jax.experimental import pallas as pl
from jax.experimental.pallas import tpu as pltpu
from jax.experimental.pallas import tpu_sc as plsc

_D = 64
_B = 4096
_H = 200
_V = 1000000
_L = 128
_NW = 32

_mesh = plsc.VectorSubcoreMesh(core_axis_name="c", subcore_axis_name="s")


@functools.partial(
    pl.kernel,
    out_type=jax.ShapeDtypeStruct((_H, _D, _B), jnp.float32),
    mesh=_mesh,
    scratch_types=[
        pltpu.VMEM((_H, _L), jnp.int32),
        pltpu.VMEM((_L,), jnp.int32),
        pltpu.VMEM((_L,), jnp.int32),
        pltpu.VMEM((_L,), jnp.int32),
        pltpu.VMEM((_L,), jnp.int32),
        pltpu.VMEM((_L, _L), jnp.float32),
        pltpu.VMEM((_L, _L), jnp.float32),
        pltpu.VMEM((_D, _L), jnp.float32),
        pltpu.VMEM((_D, _L), jnp.float32),
        pltpu.SemaphoreType.DMA,
        pltpu.SemaphoreType.DMA,
        pltpu.SemaphoreType.DMA,
        pltpu.SemaphoreType.DMA,
    ],
    compiler_params=pltpu.CompilerParams(
        use_tc_tiling_on_sc=True, needs_layout_passes=False),
)
def _gather_t(idx_hbm, tab_hbm, out_hbm, idx_v, i20, i21, hb0, hb1,
              rows0, rows1, til0, til1, sg0, sg1, sw0, sw1):
    wid = lax.axis_index("s") * 2 + lax.axis_index("c")
    b0 = wid * _L

    pltpu.sync_copy(idx_hbm.at[:, pl.ds(b0, _L)], idx_v)

    i2s = (i20, i21)
    hbs = (hb0, hb1)
    rows = (rows0, rows1)
    tils = (til0, til1)
    sgs = (sg0, sg1)
    sws = (sw0, sw1)

    lane = lax.iota(jnp.int32, 16)
    bvecs = [lane + (c * 16) for c in range(8)]

    def prep(h, p):
        for c in range(8):
            iv = idx_v[h, pl.ds(c * 16, 16)]
            i2s[p][pl.ds(c * 16, 16)] = lax.shift_right_logical(iv, 1)
            hbs[p][pl.ds(c * 16, 16)] = lax.shift_left(iv & 1, 6)

    def g_desc(p):
        return pltpu.make_async_copy(tab_hbm.at[i2s[p]], rows[p], sgs[p])

    def w_desc(h, p):
        return pltpu.make_async_copy(
            tils[p], out_hbm.at[h, :, pl.ds(b0, _L)], sws[p])

    def transpose(p):
        hvecs = [hbs[p][pl.ds(c * 16, 16)] for c in range(8)]

        @plsc.parallel_loop(0, 16, unroll=4)
        def per_s(s):
            rot = (lane + s) & 15
            for m in range(4):
                d_pure = rot + (16 * m)
                for c in range(8):
                    v = plsc.load_gather(
                        rows[p], [bvecs[c], hvecs[c] + d_pure])
                    plsc.store_scatter(tils[p], [d_pure, bvecs[c]], v)

    prep(0, 0)
    g_desc(0).start()
    prep(1, 1)
    g_desc(1).start()

    def body(j, carry):
        for p in range(2):
            h = j * 2 + p
            g_desc(p).wait()

            @pl.when(h >= 2)
            def _():
                w_desc(h - 2, p).wait()

            transpose(p)
            w_desc(h, p).start()

            @pl.when(h + 2 < _H)
            def _():
                prep(h + 2, p)
                g_desc(p).start()

        return carry

    lax.fori_loop(0, _H // 2, body, 0)
    w_desc(_H - 2, 0).wait()
    w_desc(_H - 1, 1).wait()


_NBLK = 7812
_BPW = 246


@functools.partial(
    pl.kernel,
    out_type=jax.ShapeDtypeStruct((_V // 2, 2 * _D), jnp.float32),
    mesh=_mesh,
    scratch_types=[
        pltpu.VMEM((_D, _L), jnp.float32),
        pltpu.VMEM((_D, _L), jnp.float32),
        pltpu.VMEM((_D, _L), jnp.float32),
        pltpu.VMEM((_D, _L), jnp.float32),
        pltpu.VMEM((32, _L), jnp.float32),
        pltpu.SemaphoreType.DMA,
        pltpu.SemaphoreType.DMA,
        pltpu.SemaphoreType.DMA,
        pltpu.SemaphoreType.DMA,
    ],
    compiler_params=pltpu.CompilerParams(
        use_tc_tiling_on_sc=True, needs_layout_passes=False),
)
def _pairify(tabt_hbm, tail_hbm, pairs_hbm, st0, st1, pb0, pb1, tlv,
             si0, si1, so0, so1):
    wid = lax.axis_index("s") * 2 + lax.axis_index("c")

    sts = (st0, st1)
    pbs = (pb0, pb1)
    sis = (si0, si1)
    sos = (so0, so1)

    lane = lax.iota(jnp.int32, 16)
    vvecs = [lane + (v0 * 16) for v0 in range(8)]
    qvecs = [lax.shift_right_logical(vv, 1) for vv in vvecs]
    ovecs = [lax.shift_left(vv & 1, 6) for vv in vvecs]

    def in_desc(c, p):
        return pltpu.make_async_copy(
            tabt_hbm.at[:, pl.ds(c * _L, _L)], sts[p], sis[p])

    def out_desc(c, p):
        return pltpu.make_async_copy(
            pbs[p], pairs_hbm.at[pl.ds(c * _D, _D)], sos[p])

    def fire_in(i, p):
        c = wid + i * _NW

        @pl.when(c < _NBLK)
        def _():
            in_desc(c, p).start()

    def wait_in(i, p):
        c = wid + i * _NW

        @pl.when(c < _NBLK)
        def _():
            in_desc(c, p).wait()

    def fire_out(i, p):
        c = wid + i * _NW

        @pl.when(c < _NBLK)
        def _():
            out_desc(c, p).start()

    def wait_out(i, p):
        c = wid + i * _NW

        @pl.when(c < _NBLK)
        def _():
            out_desc(c, p).wait()

    def transpose(p):
        @plsc.parallel_loop(0, 16, unroll=4)
        def per_s(s):
            rot = (lane + s) & 15
            for m in range(4):
                d_pure = rot + (16 * m)
                for v0 in range(8):
                    v = plsc.load_gather(sts[p], [d_pure, vvecs[v0]])
                    plsc.store_scatter(
                        pbs[p], [qvecs[v0], ovecs[v0] + d_pure], v)

    @pl.when(wid == 0)
    def _():
        pltpu.sync_copy(tail_hbm, tlv)
        pltpu.sync_copy(tlv, pairs_hbm.at[pl.ds(_V // 2 - 32, 32)])

    fire_in(0, 0)
    fire_in(1, 1)

    def body(j, carry):
        for p in range(2):
            i = j * 2 + p
            c = wid + i * _NW

            @pl.when(c < _NBLK)
            def _():
                wait_in(i, p)

                @pl.when(i >= 2)
                def _():
                    wait_out(i - 2, p)

                transpose(p)
                fire_out(i, p)
                fire_in(i + 2, p)

        return carry

    lax.fori_loop(0, _BPW // 2, body, 0)

    for i in (_BPW - 4, _BPW - 3, _BPW - 2, _BPW - 1):
        c = wid + i * _NW

        @pl.when((c < _NBLK) & (c + 2 * _NW >= _NBLK))
        def _():
            wait_out(i, i % 2)


def kernel(x, table):
    xt = x.astype(jnp.int32).T
    tabt = table.T
    tailp = jnp.reshape(
        lax.slice(table, (_V - _D, 0), (_V, _D)), (32, 2 * _D))
    tab2 = _pairify(tabt, tailp)
    out_t = _gather_t(xt, tab2)
    return jnp.transpose(out_t, (2, 0, 1))
